# Initial kernel scaffold; baseline (speedup 1.0000x reference)
import functools

import jax
import jax.numpy as jnp
from jax import lax
from jax.experimental import pallas as pl
from jax.experimental.pallas import tpu as pltpu
from jax.experimental.pallas import tpu_sc as plsc

_CH = 128
_NSUB = 16
_NCORE = 2
_DEGW = 16


@functools.lru_cache(maxsize=None)
def _sc_degree(n, e):
    chunks_per_core = e // _NCORE // _CH
    base_chunks = chunks_per_core // _NSUB
    extra = chunks_per_core % _NSUB
    rows_per_tile = n // _NSUB
    zr = 125
    mesh = plsc.VectorSubcoreMesh(core_axis_name="c", subcore_axis_name="s")

    @functools.partial(
        pl.kernel,
        out_type=jax.ShapeDtypeStruct((_NCORE, n, _DEGW), jnp.float32),
        mesh=mesh,
        scratch_types=[
            pltpu.VMEM((_CH,), jnp.int32),
            pltpu.VMEM((_CH, _DEGW), jnp.float32),
            pltpu.VMEM((zr, _DEGW), jnp.float32),
            pltpu.VMEM_SHARED((n, _DEGW), jnp.float32),
            pltpu.SemaphoreType.DMA,
        ],
    )
    def deg_kernel(dst_hbm, out_hbm, didx, ones, buf, table, sem):
        c = lax.axis_index("c")
        s = lax.axis_index("s")
        one_v = jnp.ones((16,), jnp.float32)
        zero_v = jnp.zeros((16,), jnp.float32)

        def fill_ones(i, _):
            ones[i, :] = one_v
            return 0

        lax.fori_loop(0, _CH, fill_ones, 0)

        def fill_zero(i, _):
            buf[i, :] = zero_v
            return 0

        lax.fori_loop(0, zr, fill_zero, 0)

        r0 = s * rows_per_tile
        for j in range(rows_per_tile // zr):
            pltpu.sync_copy(buf, table.at[pl.ds(r0 + j * zr, zr)])
        plsc.subcore_barrier()

        ebase = c * (e // _NCORE)
        my_chunks = base_chunks + (s < extra).astype(jnp.int32)

        def body(i, _):
            off = ebase + (s + i * _NSUB) * _CH
            pltpu.sync_copy(dst_hbm.at[pl.ds(off, _CH)], didx)
            pltpu.sync_copy(ones, table.at[didx], add=True)
            return 0

        lax.fori_loop(0, my_chunks, body, 0)
        plsc.subcore_barrier()

        for j in range(rows_per_tile // zr):
            pltpu.sync_copy(table.at[pl.ds(r0 + j * zr, zr)], buf)
            pltpu.sync_copy(buf, out_hbm.at[c, pl.ds(r0 + j * zr, zr)])

    return deg_kernel


@functools.lru_cache(maxsize=None)
def _sc_scatter(n, e, d):
    chunks_per_core = e // _NCORE // _CH
    base_chunks = chunks_per_core // _NSUB
    extra = chunks_per_core % _NSUB
    rows_per_tile = n // _NSUB
    zr = 125
    mesh = plsc.VectorSubcoreMesh(core_axis_name="c", subcore_axis_name="s")

    @functools.partial(
        pl.kernel,
        out_type=jax.ShapeDtypeStruct((_NCORE, n, d), jnp.float32),
        mesh=mesh,
        scratch_types=[
            pltpu.VMEM((_CH,), jnp.int32),
            pltpu.VMEM((_CH,), jnp.int32),
            pltpu.VMEM((_CH, d), jnp.float32),
            pltpu.VMEM_SHARED((n, d), jnp.float32),
            pltpu.SemaphoreType.DMA,
        ],
    )
    def scatter_kernel(y_hbm, src_hbm, dst_hbm, out_hbm, sidx, didx, rows, acc,
                       sem):
        c = lax.axis_index("c")
        s = lax.axis_index("s")
        zero_v = jnp.zeros((16,), jnp.float32)

        def fill_zero(i, _):
            def col(j, _):
                rows[i, pl.ds(j * 16, 16)] = zero_v
                return 0

            return lax.fori_loop(0, d // 16, col, 0)

        lax.fori_loop(0, zr, fill_zero, 0)

        r0 = s * rows_per_tile
        for j in range(rows_per_tile // zr):
            pltpu.sync_copy(rows.at[pl.ds(0, zr)],
                            acc.at[pl.ds(r0 + j * zr, zr)])
        plsc.subcore_barrier()

        ebase = c * (e // _NCORE)
        my_chunks = base_chunks + (s < extra).astype(jnp.int32)

        def body(i, _):
            off = ebase + (s + i * _NSUB) * _CH
            pltpu.sync_copy(src_hbm.at[pl.ds(off, _CH)], sidx)
            pltpu.sync_copy(dst_hbm.at[pl.ds(off, _CH)], didx)
            pltpu.async_copy(y_hbm.at[sidx], rows, sem).wait()
            pltpu.sync_copy(rows, acc.at[didx], add=True)
            return 0

        lax.fori_loop(0, my_chunks, body, 0)
        plsc.subcore_barrier()

        for j in range(rows_per_tile // zr):
            pltpu.sync_copy(acc.at[pl.ds(r0 + j * zr, zr)],
                            rows.at[pl.ds(0, zr)])
            pltpu.sync_copy(rows.at[pl.ds(0, zr)],
                            out_hbm.at[c, pl.ds(r0 + j * zr, zr)])

    return scatter_kernel


@functools.lru_cache(maxsize=None)
def _tc_first(n, d, blk=1000):
    def body(dp_ref, x_ref, w_ref, y_ref):
        deg = dp_ref[0, :, 0:1] + dp_ref[1, :, 0:1] + 1.0
        dinv = lax.rsqrt(deg)
        xw = jnp.dot(x_ref[...], w_ref[...],
                     preferred_element_type=jnp.float32)
        y_ref[...] = xw * dinv

    return pl.pallas_call(
        body,
        grid=(n // blk,),
        in_specs=[
            pl.BlockSpec((2, blk, _DEGW), lambda i: (0, i, 0)),
            pl.BlockSpec((blk, d), lambda i: (i, 0)),
            pl.BlockSpec((d, d), lambda i: (0, 0)),
        ],
        out_specs=pl.BlockSpec((blk, d), lambda i: (i, 0)),
        out_shape=jax.ShapeDtypeStruct((n, d), jnp.float32),
    )


@functools.lru_cache(maxsize=None)
def _tc_mid(n, d, blk=1000):
    def body(dp_ref, acc_ref, y_ref, b_ref, w_ref, out_ref):
        deg = dp_ref[0, :, 0:1] + dp_ref[1, :, 0:1] + 1.0
        dinv = lax.rsqrt(deg)
        t = (acc_ref[0] + acc_ref[1] + y_ref[...]) * dinv + b_ref[...][None, :]
        h = jnp.maximum(t, 0.0)
        hw = jnp.dot(h, w_ref[...], preferred_element_type=jnp.float32)
        out_ref[...] = hw * dinv

    return pl.pallas_call(
        body,
        grid=(n // blk,),
        in_specs=[
            pl.BlockSpec((2, blk, _DEGW), lambda i: (0, i, 0)),
            pl.BlockSpec((2, blk, d), lambda i: (0, i, 0)),
            pl.BlockSpec((blk, d), lambda i: (i, 0)),
            pl.BlockSpec((d,), lambda i: (0,)),
            pl.BlockSpec((d, d), lambda i: (0, 0)),
        ],
        out_specs=pl.BlockSpec((blk, d), lambda i: (i, 0)),
        out_shape=jax.ShapeDtypeStruct((n, d), jnp.float32),
    )


@functools.lru_cache(maxsize=None)
def _tc_last(n, d, blk=1000):
    def body(dp_ref, acc_ref, y_ref, b_ref, out_ref):
        deg = dp_ref[0, :, 0:1] + dp_ref[1, :, 0:1] + 1.0
        dinv = lax.rsqrt(deg)
        out_ref[...] = ((acc_ref[0] + acc_ref[1] + y_ref[...]) * dinv
                        + b_ref[...][None, :])

    return pl.pallas_call(
        body,
        grid=(n // blk,),
        in_specs=[
            pl.BlockSpec((2, blk, _DEGW), lambda i: (0, i, 0)),
            pl.BlockSpec((2, blk, d), lambda i: (0, i, 0)),
            pl.BlockSpec((blk, d), lambda i: (i, 0)),
            pl.BlockSpec((d,), lambda i: (0,)),
        ],
        out_specs=pl.BlockSpec((blk, d), lambda i: (i, 0)),
        out_shape=jax.ShapeDtypeStruct((n, d), jnp.float32),
    )


def kernel(x, edge_index, W1, b1, W2, b2, W3, b3):
    n, d = x.shape
    e = edge_index.shape[1]
    assert n % _NSUB == 0 and (n // _NSUB) % 125 == 0
    assert e % (_NCORE * _CH) == 0 and d % 16 == 0

    src = edge_index[0]
    dst = edge_index[1]

    degparts = _sc_degree(n, e)(dst)
    scatter = _sc_scatter(n, e, d)

    y1 = _tc_first(n, d)(degparts, x, W1)
    acc1 = scatter(y1, src, dst)
    y2 = _tc_mid(n, d)(degparts, acc1, y1, b1, W2)
    acc2 = scatter(y2, src, dst)
    y3 = _tc_mid(n, d)(degparts, acc2, y2, b2, W3)
    acc3 = scatter(y3, src, dst)
    return _tc_last(n, d)(degparts, acc3, y3, b3)


# speedup vs baseline: 13.8397x; 13.8397x over previous
import functools

import jax
import jax.numpy as jnp
from jax import lax
from jax.experimental import pallas as pl
from jax.experimental.pallas import tpu as pltpu
from jax.experimental.pallas import tpu_sc as plsc

_CH = 128
_NSUB = 16
_NCORE = 2
_DEGW = 16


@functools.lru_cache(maxsize=None)
def _sc_degree(n, e):
    chunks_per_core = e // _NCORE // _CH
    base_chunks = chunks_per_core // _NSUB
    extra = chunks_per_core % _NSUB
    zr = 200
    nz = n // zr
    nz_rounds = -(-nz // _NSUB)
    mesh = plsc.VectorSubcoreMesh(core_axis_name="c", subcore_axis_name="s")

    @functools.partial(
        pl.kernel,
        out_type=jax.ShapeDtypeStruct((_NCORE, n, _DEGW), jnp.float32),
        mesh=mesh,
        scratch_types=[
            pltpu.VMEM((_CH,), jnp.int32),
            pltpu.VMEM((_CH, _DEGW), jnp.float32),
            pltpu.VMEM((zr, _DEGW), jnp.float32),
            pltpu.VMEM_SHARED((n, _DEGW), jnp.float32),
            pltpu.SemaphoreType.DMA,
        ],
    )
    def deg_kernel(dst_hbm, out_hbm, didx, ones, buf, table, sem):
        c = lax.axis_index("c")
        s = lax.axis_index("s")
        one_v = jnp.ones((16,), jnp.float32)
        zero_v = jnp.zeros((16,), jnp.float32)

        def fill_ones(i, _):
            ones[i, :] = one_v
            return 0

        lax.fori_loop(0, _CH, fill_ones, 0)

        def fill_zero(i, _):
            buf[i, :] = zero_v
            return 0

        lax.fori_loop(0, zr, fill_zero, 0)

        for j in range(nz_rounds):
            ch = s + j * _NSUB

            @pl.when(ch < nz)
            def _():
                pltpu.sync_copy(buf, table.at[pl.ds(ch * zr, zr)])

        plsc.subcore_barrier()

        ebase = c * (e // _NCORE)
        my_chunks = base_chunks + (s < extra).astype(jnp.int32)

        def body(i, _):
            off = ebase + (s + i * _NSUB) * _CH
            pltpu.sync_copy(dst_hbm.at[pl.ds(off, _CH)], didx)
            pltpu.sync_copy(ones, table.at[didx], add=True)
            return 0

        lax.fori_loop(0, my_chunks, body, 0)
        plsc.subcore_barrier()

        for j in range(nz_rounds):
            ch = s + j * _NSUB

            @pl.when(ch < nz)
            def _():
                pltpu.sync_copy(table.at[pl.ds(ch * zr, zr)], buf)
                pltpu.sync_copy(buf, out_hbm.at[c, pl.ds(ch * zr, zr)])

    return deg_kernel


@functools.lru_cache(maxsize=None)
def _sc_scatter(n, e, d):
    chunks_per_core = e // _NCORE // _CH
    base_chunks = chunks_per_core // _NSUB
    extra = chunks_per_core % _NSUB
    zr = 200
    nz = n // zr
    nz_rounds = -(-nz // _NSUB)
    mesh = plsc.VectorSubcoreMesh(core_axis_name="c", subcore_axis_name="s")

    @functools.partial(
        pl.kernel,
        out_type=jax.ShapeDtypeStruct((_NCORE, n, d), jnp.float32),
        mesh=mesh,
        scratch_types=[
            pltpu.VMEM((_CH,), jnp.int32),
            pltpu.VMEM((_CH,), jnp.int32),
            pltpu.VMEM((zr, d), jnp.float32),
            pltpu.VMEM_SHARED((n, d), jnp.float32),
            pltpu.SemaphoreType.DMA,
        ],
    )
    def scatter_kernel(y_hbm, src_hbm, dst_hbm, out_hbm, sidx, didx, rows, acc,
                       sem):
        c = lax.axis_index("c")
        s = lax.axis_index("s")
        zero_v = jnp.zeros((16,), jnp.float32)

        def fill_zero(i, _):
            def col(j, _):
                rows[i, pl.ds(j * 16, 16)] = zero_v
                return 0

            return lax.fori_loop(0, d // 16, col, 0)

        lax.fori_loop(0, zr, fill_zero, 0)

        for j in range(nz_rounds):
            ch = s + j * _NSUB

            @pl.when(ch < nz)
            def _():
                pltpu.sync_copy(rows, acc.at[pl.ds(ch * zr, zr)])

        plsc.subcore_barrier()

        ebase = c * (e // _NCORE)
        my_chunks = base_chunks + (s < extra).astype(jnp.int32)

        def body(i, _):
            off = ebase + (s + i * _NSUB) * _CH
            pltpu.sync_copy(src_hbm.at[pl.ds(off, _CH)], sidx)
            pltpu.sync_copy(dst_hbm.at[pl.ds(off, _CH)], didx)
            pltpu.async_copy(y_hbm.at[sidx], rows.at[pl.ds(0, _CH)],
                             sem).wait()
            pltpu.sync_copy(rows.at[pl.ds(0, _CH)], acc.at[didx], add=True)
            return 0

        lax.fori_loop(0, my_chunks, body, 0)
        plsc.subcore_barrier()

        for j in range(nz_rounds):
            ch = s + j * _NSUB

            @pl.when(ch < nz)
            def _():
                pltpu.sync_copy(acc.at[pl.ds(ch * zr, zr)], rows)
                pltpu.sync_copy(rows, out_hbm.at[c, pl.ds(ch * zr, zr)])

    return scatter_kernel


@functools.lru_cache(maxsize=None)
def _tc_first(n, d, blk=1000):
    def body(dp_ref, x_ref, w_ref, y_ref):
        deg = dp_ref[0, :, 0:1] + dp_ref[1, :, 0:1] + 1.0
        dinv = lax.rsqrt(deg)
        xw = jnp.dot(x_ref[...], w_ref[...],
                     preferred_element_type=jnp.float32)
        y_ref[...] = xw * dinv

    return pl.pallas_call(
        body,
        grid=(n // blk,),
        in_specs=[
            pl.BlockSpec((2, blk, _DEGW), lambda i: (0, i, 0)),
            pl.BlockSpec((blk, d), lambda i: (i, 0)),
            pl.BlockSpec((d, d), lambda i: (0, 0)),
        ],
        out_specs=pl.BlockSpec((blk, d), lambda i: (i, 0)),
        out_shape=jax.ShapeDtypeStruct((n, d), jnp.float32),
    )


@functools.lru_cache(maxsize=None)
def _tc_mid(n, d, blk=1000):
    def body(dp_ref, acc_ref, y_ref, b_ref, w_ref, out_ref):
        deg = dp_ref[0, :, 0:1] + dp_ref[1, :, 0:1] + 1.0
        dinv = lax.rsqrt(deg)
        t = (acc_ref[0] + acc_ref[1] + y_ref[...]) * dinv + b_ref[...][None, :]
        h = jnp.maximum(t, 0.0)
        hw = jnp.dot(h, w_ref[...], preferred_element_type=jnp.float32)
        out_ref[...] = hw * dinv

    return pl.pallas_call(
        body,
        grid=(n // blk,),
        in_specs=[
            pl.BlockSpec((2, blk, _DEGW), lambda i: (0, i, 0)),
            pl.BlockSpec((2, blk, d), lambda i: (0, i, 0)),
            pl.BlockSpec((blk, d), lambda i: (i, 0)),
            pl.BlockSpec((d,), lambda i: (0,)),
            pl.BlockSpec((d, d), lambda i: (0, 0)),
        ],
        out_specs=pl.BlockSpec((blk, d), lambda i: (i, 0)),
        out_shape=jax.ShapeDtypeStruct((n, d), jnp.float32),
    )


@functools.lru_cache(maxsize=None)
def _tc_last(n, d, blk=1000):
    def body(dp_ref, acc_ref, y_ref, b_ref, out_ref):
        deg = dp_ref[0, :, 0:1] + dp_ref[1, :, 0:1] + 1.0
        dinv = lax.rsqrt(deg)
        out_ref[...] = ((acc_ref[0] + acc_ref[1] + y_ref[...]) * dinv
                        + b_ref[...][None, :])

    return pl.pallas_call(
        body,
        grid=(n // blk,),
        in_specs=[
            pl.BlockSpec((2, blk, _DEGW), lambda i: (0, i, 0)),
            pl.BlockSpec((2, blk, d), lambda i: (0, i, 0)),
            pl.BlockSpec((blk, d), lambda i: (i, 0)),
            pl.BlockSpec((d,), lambda i: (0,)),
        ],
        out_specs=pl.BlockSpec((blk, d), lambda i: (i, 0)),
        out_shape=jax.ShapeDtypeStruct((n, d), jnp.float32),
    )


def kernel(x, edge_index, W1, b1, W2, b2, W3, b3):
    n, d = x.shape
    e = edge_index.shape[1]
    assert n % 200 == 0
    assert e % (_NCORE * _CH) == 0 and d % 16 == 0

    src = edge_index[0]
    dst = edge_index[1]

    degparts = _sc_degree(n, e)(dst)
    scatter = _sc_scatter(n, e, d)

    y1 = _tc_first(n, d)(degparts, x, W1)
    acc1 = scatter(y1, src, dst)
    y2 = _tc_mid(n, d)(degparts, acc1, y1, b1, W2)
    acc2 = scatter(y2, src, dst)
    y3 = _tc_mid(n, d)(degparts, acc2, y2, b2, W3)
    acc3 = scatter(y3, src, dst)
    return _tc_last(n, d)(degparts, acc3, y3, b3)
